# baseline (speedup 1.0000x reference)
import functools
import math

import jax
import jax.numpy as jnp
import numpy as np
from jax import lax
from jax.experimental import pallas as pl
from jax.experimental.pallas import tpu as pltpu
from jax.experimental.pallas import tpu_sc as plsc

N = 65536
NCHUNK = 32
CH = 2048
LANES = 16


def _rev_bits(x, nbits):
    r = np.zeros_like(x)
    t = x.copy()
    for _ in range(nbits):
        r = (r << 1) | (t & 1)
        t >>= 1
    return r

_rev16 = _rev_bits(np.arange(N, dtype=np.int64), 16)
_BITREV_IDX = np.stack([2 * _rev16, 2 * _rev16 + 1], axis=0).reshape(
    2, NCHUNK, LANES, 128).transpose(1, 0, 2, 3).astype(np.int32).copy()

_lane = np.arange(LANES, dtype=np.int64)
_wr163, _wi163 = [], []
for _s in range(1, 4):
    _h = 1 << _s
    _a = -2.0 * np.pi * (_lane & (_h - 1)) / (2 * _h)
    _wr163.append(np.cos(_a))
    _wi163.append(np.sin(_a))
_LOC_OFF = {}
_twc, _tws = [], []
_o = 0
for _s in range(4, 11):
    _h = 1 << _s
    _a = -2.0 * np.pi * np.arange(_h, dtype=np.float64) / (2 * _h)
    _twc.append(np.cos(_a))
    _tws.append(np.sin(_a))
    _LOC_OFF[_s] = _o
    _o += _h
_NLOC = _o
_WR163_OFF = 0
_WI163_OFF = 48
_TWC_OFF = 96
_TWS_OFF = 96 + _NLOC
_NCONST = 96 + 2 * _NLOC
_CONSTS = np.concatenate(_wr163 + _wi163 + _twc + _tws).astype(np.float32)
assert _CONSTS.shape == (_NCONST,)

_XOFF = {}
_xwr, _xwi = [], []
_o = 0
for _q in range(4):
    _XOFF[_q] = _o
    _n = (1 << _q) * CH
    _a = -2.0 * np.pi * np.arange(_n, dtype=np.float64) / (1 << (12 + _q))
    _xwr.append(np.cos(_a))
    _xwi.append(np.sin(_a))
    _o += _n
_XWR = np.concatenate(_xwr).astype(np.float32)
_XWI = np.concatenate(_xwi).astype(np.float32)

_a15 = -2.0 * np.pi * np.arange(N // 2, dtype=np.float64) / N
_WR15 = np.cos(_a15).astype(np.float32)
_WI15 = np.sin(_a15).astype(np.float32)

_MESH = plsc.VectorSubcoreMesh(
    core_axis_name="c", subcore_axis_name="s", num_cores=2, num_subcores=16)


def _k1_body(x_hbm, idx_hbm, consts_hbm, xwr_hbm, xwi_hbm,
             ore_hbm, oim_hbm,
             idx_v, re_v, im_v, tw_v, pre_v, pim_v, xwr_v, xwi_v,
             shr_re, shr_im, sem):
    sid = lax.axis_index("s")
    wid = lax.axis_index("c") * 16 + sid

    pltpu.sync_copy(idx_hbm.at[wid], idx_v)
    copies = []
    for j in range(LANES):
        d = pl.ds(j * 128, 128)
        copies.append(pltpu.make_async_copy(
            x_hbm.at[idx_v.at[0, j]], re_v.at[d], sem))
        copies.append(pltpu.make_async_copy(
            x_hbm.at[idx_v.at[1, j]], im_v.at[d], sem))
    for c in copies:
        c.start()
    pltpu.sync_copy(consts_hbm, tw_v)
    for q in range(4):
        off = pl.multiple_of(_XOFF[q] + (sid & ((1 << q) - 1)) * CH, CH)
        pltpu.sync_copy(xwr_hbm.at[pl.ds(off, CH)],
                        xwr_v.at[pl.ds(q * CH, CH)])
        pltpu.sync_copy(xwi_hbm.at[pl.ds(off, CH)],
                        xwi_v.at[pl.ds(q * CH, CH)])
    for c in copies:
        c.wait()

    iota = lax.iota(jnp.int32, LANES)

    for s in range(0, 4):
        h = 1 << s
        pat = ((iota >> s) << (s + 1)) + (iota & (h - 1))
        if s > 0:
            wr = tw_v[pl.ds(_WR163_OFF + (s - 1) * 16, 16)]
            wi = tw_v[pl.ds(_WI163_OFF + (s - 1) * 16, 16)]

        def body03(k, s=s, h=h, pat=pat,
                   wr=(None if s == 0 else wr), wi=(None if s == 0 else wi)):
            ti = k * 32 + pat
            bi_ = ti + h
            tr = plsc.load_gather(re_v, [ti])
            tii = plsc.load_gather(im_v, [ti])
            br = plsc.load_gather(re_v, [bi_])
            bii = plsc.load_gather(im_v, [bi_])
            if s == 0:
                xr, xi = br, bii
            else:
                xr = wr * br - wi * bii
                xi = wi * br + wr * bii
            plsc.store_scatter(re_v, [ti], tr + xr)
            plsc.store_scatter(im_v, [ti], tii + xi)
            plsc.store_scatter(re_v, [bi_], tr - xr)
            plsc.store_scatter(im_v, [bi_], tii - xi)

        plsc.parallel_loop(0, 64, 1, unroll=4)(body03)

    for s in (4, 6, 8):
        h = 1 << s

        def body4(q, s=s, h=h):
            r = q & (h - 1)
            i0 = ((q >> s) << (s + 2)) + r
            wr = tw_v[pl.ds(_TWC_OFF + _LOC_OFF[s] + r, 16)]
            wi = tw_v[pl.ds(_TWS_OFF + _LOC_OFF[s] + r, 16)]
            u0r = tw_v[pl.ds(_TWC_OFF + _LOC_OFF[s + 1] + r, 16)]
            u0i = tw_v[pl.ds(_TWS_OFF + _LOC_OFF[s + 1] + r, 16)]
            u1r = tw_v[pl.ds(_TWC_OFF + _LOC_OFF[s + 1] + r + h, 16)]
            u1i = tw_v[pl.ds(_TWS_OFF + _LOC_OFF[s + 1] + r + h, 16)]
            d0 = pl.ds(i0, 16)
            d1 = pl.ds(i0 + h, 16)
            d2 = pl.ds(i0 + 2 * h, 16)
            d3 = pl.ds(i0 + 3 * h, 16)
            a0r, a0i = re_v[d0], im_v[d0]
            a1r, a1i = re_v[d1], im_v[d1]
            a2r, a2i = re_v[d2], im_v[d2]
            a3r, a3i = re_v[d3], im_v[d3]
            x1r = wr * a1r - wi * a1i
            x1i = wi * a1r + wr * a1i
            b0r, b0i = a0r + x1r, a0i + x1i
            b1r, b1i = a0r - x1r, a0i - x1i
            x3r = wr * a3r - wi * a3i
            x3i = wi * a3r + wr * a3i
            b2r, b2i = a2r + x3r, a2i + x3i
            b3r, b3i = a2r - x3r, a2i - x3i
            y2r = u0r * b2r - u0i * b2i
            y2i = u0i * b2r + u0r * b2i
            re_v[d0], im_v[d0] = b0r + y2r, b0i + y2i
            re_v[d2], im_v[d2] = b0r - y2r, b0i - y2i
            y3r = u1r * b3r - u1i * b3i
            y3i = u1i * b3r + u1r * b3i
            re_v[d1], im_v[d1] = b1r + y3r, b1i + y3i
            re_v[d3], im_v[d3] = b1r - y3r, b1i - y3i

        plsc.parallel_loop(0, CH // 4, 16, unroll=2)(body4)

    s10, h10 = 10, 1 << 10

    def body10(b):
        r = b & (h10 - 1)
        t0 = ((b >> s10) << (s10 + 1)) + r
        b0 = t0 + h10
        wr = tw_v[pl.ds(_TWC_OFF + _LOC_OFF[s10] + r, 16)]
        wi = tw_v[pl.ds(_TWS_OFF + _LOC_OFF[s10] + r, 16)]
        tr = re_v[pl.ds(t0, 16)]
        tii = im_v[pl.ds(t0, 16)]
        br = re_v[pl.ds(b0, 16)]
        bii = im_v[pl.ds(b0, 16)]
        xr = wr * br - wi * bii
        xi = wi * br + wr * bii
        re_v[pl.ds(t0, 16)] = tr + xr
        im_v[pl.ds(t0, 16)] = tii + xi
        re_v[pl.ds(b0, 16)] = tr - xr
        im_v[pl.ds(b0, 16)] = tii - xi

    plsc.parallel_loop(0, CH // 2, 16, unroll=4)(body10)

    pltpu.sync_copy(re_v, shr_re.at[sid])
    pltpu.sync_copy(im_v, shr_im.at[sid])
    plsc.subcore_barrier()

    for q in range(4):
        psid = sid ^ (1 << q)
        b = q & 1
        pltpu.sync_copy(shr_re.at[b * 16 + psid], pre_v)
        pltpu.sync_copy(shr_im.at[b * 16 + psid], pim_v)
        mt = (((sid >> q) & 1) ^ 1).astype(jnp.float32)
        pt = 1.0 - mt
        sign = 2.0 * mt - 1.0

        def bodyx(o, q=q, mt=mt, pt=pt, sign=sign):
            d = pl.ds(o, 16)
            mr = re_v[d]
            mi = im_v[d]
            pr = pre_v[d]
            pi = pim_v[d]
            wr = xwr_v[pl.ds(q * CH + o, 16)]
            wi = xwi_v[pl.ds(q * CH + o, 16)]
            tr = mt * mr + pt * pr
            tii = mt * mi + pt * pi
            br = mt * pr + pt * mr
            bii = mt * pi + pt * mi
            xr = wr * br - wi * bii
            xi = wi * br + wr * bii
            re_v[d] = tr + sign * xr
            im_v[d] = tii + sign * xi

        plsc.parallel_loop(0, CH, 16, unroll=4)(bodyx)
        if q < 3:
            nb = (q + 1) & 1
            pltpu.sync_copy(re_v, shr_re.at[nb * 16 + sid])
            pltpu.sync_copy(im_v, shr_im.at[nb * 16 + sid])
            plsc.subcore_barrier()

    base = pl.multiple_of(wid * CH, CH)
    pltpu.sync_copy(re_v, ore_hbm.at[pl.ds(base, CH)])
    pltpu.sync_copy(im_v, oim_hbm.at[pl.ds(base, CH)])


_k1 = functools.partial(
    pl.kernel,
    out_type=(jax.ShapeDtypeStruct((N,), jnp.float32),
              jax.ShapeDtypeStruct((N,), jnp.float32)),
    mesh=_MESH,
    compiler_params=pltpu.CompilerParams(needs_layout_passes=False),
    scratch_types=[
        pltpu.VMEM((2, LANES, 128), jnp.int32),
        pltpu.VMEM((CH,), jnp.float32),
        pltpu.VMEM((CH,), jnp.float32),
        pltpu.VMEM((_NCONST,), jnp.float32),
        pltpu.VMEM((CH,), jnp.float32),
        pltpu.VMEM((CH,), jnp.float32),
        pltpu.VMEM((4 * CH,), jnp.float32),
        pltpu.VMEM((4 * CH,), jnp.float32),
        pltpu.VMEM_SHARED((32, CH), jnp.float32),
        pltpu.VMEM_SHARED((32, CH), jnp.float32),
        pltpu.SemaphoreType.DMA,
    ],
)(_k1_body)


_NB = N // 2 // NCHUNK
_S15 = 15
_H15 = 1 << _S15


def _s15_body(re_hbm, im_hbm, wr_hbm, wi_hbm, out_hbm,
              tre, tim, bre, bim, twr, twi, obuf):
    wid = lax.axis_index("c") * 16 + lax.axis_index("s")
    b0 = wid * _NB
    t0 = pl.multiple_of(((b0 >> _S15) << (_S15 + 1)) + (b0 & (_H15 - 1)), _NB)
    j0 = pl.multiple_of(b0 & (_H15 - 1), _NB)

    pltpu.sync_copy(re_hbm.at[pl.ds(t0, _NB)], tre)
    pltpu.sync_copy(im_hbm.at[pl.ds(t0, _NB)], tim)
    pltpu.sync_copy(re_hbm.at[pl.ds(t0 + _H15, _NB)], bre)
    pltpu.sync_copy(im_hbm.at[pl.ds(t0 + _H15, _NB)], bim)
    pltpu.sync_copy(wr_hbm.at[pl.ds(j0, _NB)], twr)
    pltpu.sync_copy(wi_hbm.at[pl.ds(j0, _NB)], twi)

    def body(o):
        d = pl.ds(o, 16)
        tr = tre[d]
        tii = tim[d]
        br = bre[d]
        bii = bim[d]
        wr = twr[d]
        wi = twi[d]
        xr = wr * br - wi * bii
        xi = wi * br + wr * bii
        tre[d] = tr + xr
        tim[d] = tii + xi
        bre[d] = tr - xr
        bim[d] = tii - xi

    plsc.parallel_loop(0, _NB, 16, unroll=4)(body)

    iota2 = lax.iota(jnp.int32, 16) * 2

    def mk_inter(src_r, src_i):
        def inter(o):
            d = pl.ds(o, 16)
            ix = 2 * o + iota2
            plsc.store_scatter(obuf, [ix], src_r[d])
            plsc.store_scatter(obuf, [ix + 1], src_i[d])
        return inter

    plsc.parallel_loop(0, _NB, 16, unroll=4)(mk_inter(tre, tim))
    pltpu.sync_copy(obuf, out_hbm.at[pl.ds(2 * t0, 2 * _NB)])
    plsc.parallel_loop(0, _NB, 16, unroll=4)(mk_inter(bre, bim))
    pltpu.sync_copy(obuf, out_hbm.at[pl.ds(2 * (t0 + _H15), 2 * _NB)])


_s15 = functools.partial(
    pl.kernel,
    out_type=jax.ShapeDtypeStruct((2 * N,), jnp.float32),
    mesh=_MESH,
    compiler_params=pltpu.CompilerParams(needs_layout_passes=False),
    scratch_types=[pltpu.VMEM((_NB,), jnp.float32)] * 6
    + [pltpu.VMEM((2 * _NB,), jnp.float32)],
)(_s15_body)


def kernel(x):
    re, im = _k1(x.reshape(-1), jnp.asarray(_BITREV_IDX),
                 jnp.asarray(_CONSTS),
                 jnp.asarray(_XWR), jnp.asarray(_XWI))
    out = _s15(re, im, jnp.asarray(_WR15), jnp.asarray(_WI15))
    return out.reshape(N, 2)


# speedup vs baseline: 1.3152x; 1.3152x over previous
import functools
import math

import jax
import jax.numpy as jnp
import numpy as np
from jax import lax
from jax.experimental import pallas as pl
from jax.experimental.pallas import tpu as pltpu
from jax.experimental.pallas import tpu_sc as plsc

N = 65536
NCHUNK = 32
CH = 2048
LANES = 16


def _rev_bits(x, nbits):
    r = np.zeros_like(x)
    t = x.copy()
    for _ in range(nbits):
        r = (r << 1) | (t & 1)
        t >>= 1
    return r

_BITREV_IDX = _rev_bits(np.arange(N, dtype=np.int64), 16).reshape(
    NCHUNK, LANES, 128).astype(np.int32)

_lane = np.arange(LANES, dtype=np.int64)
_wr163, _wi163 = [], []
for _s in range(1, 4):
    _h = 1 << _s
    _a = -2.0 * np.pi * (_lane & (_h - 1)) / (2 * _h)
    _wr163.append(np.cos(_a))
    _wi163.append(np.sin(_a))
_LOC_OFF = {}
_twc, _tws = [], []
_o = 0
for _s in range(4, 11):
    _h = 1 << _s
    _a = -2.0 * np.pi * np.arange(_h, dtype=np.float64) / (2 * _h)
    _twc.append(np.cos(_a))
    _tws.append(np.sin(_a))
    _LOC_OFF[_s] = _o
    _o += _h
_NLOC = _o
_WR163_OFF = 0
_WI163_OFF = 48
_TWC_OFF = 96
_TWS_OFF = 96 + _NLOC
_NCONST = 96 + 2 * _NLOC
_CONSTS = np.concatenate(_wr163 + _wi163 + _twc + _tws).astype(np.float32)
assert _CONSTS.shape == (_NCONST,)

_XOFF = {}
_xwr, _xwi = [], []
_o = 0
for _q in range(4):
    _XOFF[_q] = _o
    _n = (1 << _q) * CH
    _a = -2.0 * np.pi * np.arange(_n, dtype=np.float64) / (1 << (12 + _q))
    _xwr.append(np.cos(_a))
    _xwi.append(np.sin(_a))
    _o += _n
_XWR = np.concatenate(_xwr).astype(np.float32)
_XWI = np.concatenate(_xwi).astype(np.float32)

_a15 = -2.0 * np.pi * np.arange(N // 2, dtype=np.float64) / N
_WR15 = np.cos(_a15).astype(np.float32)
_WI15 = np.sin(_a15).astype(np.float32)

_MESH = plsc.VectorSubcoreMesh(
    core_axis_name="c", subcore_axis_name="s", num_cores=2, num_subcores=16)


def _k1_body(re_hbm, im_hbm, idx_hbm, consts_hbm, xwr_hbm, xwi_hbm,
             ore_hbm, oim_hbm,
             idx_v, re_v, im_v, tw_v, pre_v, pim_v, xwr_v, xwi_v,
             shr_re, shr_im, sem):
    sid = lax.axis_index("s")
    wid = lax.axis_index("c") * 16 + sid

    pltpu.sync_copy(idx_hbm.at[wid], idx_v)
    copies = []
    for j in range(LANES):
        d = pl.ds(j * 128, 128)
        copies.append(pltpu.make_async_copy(
            re_hbm.at[idx_v.at[j]], re_v.at[d], sem))
        copies.append(pltpu.make_async_copy(
            im_hbm.at[idx_v.at[j]], im_v.at[d], sem))
    for c in copies:
        c.start()
    pltpu.sync_copy(consts_hbm, tw_v)
    for q in range(4):
        off = pl.multiple_of(_XOFF[q] + (sid & ((1 << q) - 1)) * CH, CH)
        pltpu.sync_copy(xwr_hbm.at[pl.ds(off, CH)],
                        xwr_v.at[pl.ds(q * CH, CH)])
        pltpu.sync_copy(xwi_hbm.at[pl.ds(off, CH)],
                        xwi_v.at[pl.ds(q * CH, CH)])
    for c in copies:
        c.wait()

    iota = lax.iota(jnp.int32, LANES)

    for s in range(0, 4):
        h = 1 << s
        pat = ((iota >> s) << (s + 1)) + (iota & (h - 1))
        if s > 0:
            wr = tw_v[pl.ds(_WR163_OFF + (s - 1) * 16, 16)]
            wi = tw_v[pl.ds(_WI163_OFF + (s - 1) * 16, 16)]

        def body03(k, s=s, h=h, pat=pat,
                   wr=(None if s == 0 else wr), wi=(None if s == 0 else wi)):
            ti = k * 32 + pat
            bi_ = ti + h
            tr = plsc.load_gather(re_v, [ti])
            tii = plsc.load_gather(im_v, [ti])
            br = plsc.load_gather(re_v, [bi_])
            bii = plsc.load_gather(im_v, [bi_])
            if s == 0:
                xr, xi = br, bii
            else:
                xr = wr * br - wi * bii
                xi = wi * br + wr * bii
            plsc.store_scatter(re_v, [ti], tr + xr)
            plsc.store_scatter(im_v, [ti], tii + xi)
            plsc.store_scatter(re_v, [bi_], tr - xr)
            plsc.store_scatter(im_v, [bi_], tii - xi)

        plsc.parallel_loop(0, 64, 1, unroll=4)(body03)

    for s in (4, 6, 8):
        h = 1 << s

        def body4(q, s=s, h=h):
            r = q & (h - 1)
            i0 = ((q >> s) << (s + 2)) + r
            wr = tw_v[pl.ds(_TWC_OFF + _LOC_OFF[s] + r, 16)]
            wi = tw_v[pl.ds(_TWS_OFF + _LOC_OFF[s] + r, 16)]
            u0r = tw_v[pl.ds(_TWC_OFF + _LOC_OFF[s + 1] + r, 16)]
            u0i = tw_v[pl.ds(_TWS_OFF + _LOC_OFF[s + 1] + r, 16)]
            u1r = tw_v[pl.ds(_TWC_OFF + _LOC_OFF[s + 1] + r + h, 16)]
            u1i = tw_v[pl.ds(_TWS_OFF + _LOC_OFF[s + 1] + r + h, 16)]
            d0 = pl.ds(i0, 16)
            d1 = pl.ds(i0 + h, 16)
            d2 = pl.ds(i0 + 2 * h, 16)
            d3 = pl.ds(i0 + 3 * h, 16)
            a0r, a0i = re_v[d0], im_v[d0]
            a1r, a1i = re_v[d1], im_v[d1]
            a2r, a2i = re_v[d2], im_v[d2]
            a3r, a3i = re_v[d3], im_v[d3]
            x1r = wr * a1r - wi * a1i
            x1i = wi * a1r + wr * a1i
            b0r, b0i = a0r + x1r, a0i + x1i
            b1r, b1i = a0r - x1r, a0i - x1i
            x3r = wr * a3r - wi * a3i
            x3i = wi * a3r + wr * a3i
            b2r, b2i = a2r + x3r, a2i + x3i
            b3r, b3i = a2r - x3r, a2i - x3i
            y2r = u0r * b2r - u0i * b2i
            y2i = u0i * b2r + u0r * b2i
            re_v[d0], im_v[d0] = b0r + y2r, b0i + y2i
            re_v[d2], im_v[d2] = b0r - y2r, b0i - y2i
            y3r = u1r * b3r - u1i * b3i
            y3i = u1i * b3r + u1r * b3i
            re_v[d1], im_v[d1] = b1r + y3r, b1i + y3i
            re_v[d3], im_v[d3] = b1r - y3r, b1i - y3i

        plsc.parallel_loop(0, CH // 4, 16, unroll=2)(body4)

    s10, h10 = 10, 1 << 10

    def body10(b):
        r = b & (h10 - 1)
        t0 = ((b >> s10) << (s10 + 1)) + r
        b0 = t0 + h10
        wr = tw_v[pl.ds(_TWC_OFF + _LOC_OFF[s10] + r, 16)]
        wi = tw_v[pl.ds(_TWS_OFF + _LOC_OFF[s10] + r, 16)]
        tr = re_v[pl.ds(t0, 16)]
        tii = im_v[pl.ds(t0, 16)]
        br = re_v[pl.ds(b0, 16)]
        bii = im_v[pl.ds(b0, 16)]
        xr = wr * br - wi * bii
        xi = wi * br + wr * bii
        re_v[pl.ds(t0, 16)] = tr + xr
        im_v[pl.ds(t0, 16)] = tii + xi
        re_v[pl.ds(b0, 16)] = tr - xr
        im_v[pl.ds(b0, 16)] = tii - xi

    plsc.parallel_loop(0, CH // 2, 16, unroll=4)(body10)

    pltpu.sync_copy(re_v, shr_re.at[sid])
    pltpu.sync_copy(im_v, shr_im.at[sid])
    plsc.subcore_barrier()

    for q in range(4):
        psid = sid ^ (1 << q)
        b = q & 1
        pltpu.sync_copy(shr_re.at[b * 16 + psid], pre_v)
        pltpu.sync_copy(shr_im.at[b * 16 + psid], pim_v)
        mt = (((sid >> q) & 1) ^ 1).astype(jnp.float32)
        pt = 1.0 - mt
        sign = 2.0 * mt - 1.0

        def bodyx(o, q=q, mt=mt, pt=pt, sign=sign):
            d = pl.ds(o, 16)
            mr = re_v[d]
            mi = im_v[d]
            pr = pre_v[d]
            pi = pim_v[d]
            wr = xwr_v[pl.ds(q * CH + o, 16)]
            wi = xwi_v[pl.ds(q * CH + o, 16)]
            tr = mt * mr + pt * pr
            tii = mt * mi + pt * pi
            br = mt * pr + pt * mr
            bii = mt * pi + pt * mi
            xr = wr * br - wi * bii
            xi = wi * br + wr * bii
            re_v[d] = tr + sign * xr
            im_v[d] = tii + sign * xi

        plsc.parallel_loop(0, CH, 16, unroll=4)(bodyx)
        if q < 3:
            nb = (q + 1) & 1
            pltpu.sync_copy(re_v, shr_re.at[nb * 16 + sid])
            pltpu.sync_copy(im_v, shr_im.at[nb * 16 + sid])
            plsc.subcore_barrier()

    base = pl.multiple_of(wid * CH, CH)
    pltpu.sync_copy(re_v, ore_hbm.at[pl.ds(base, CH)])
    pltpu.sync_copy(im_v, oim_hbm.at[pl.ds(base, CH)])


_k1 = functools.partial(
    pl.kernel,
    out_type=(jax.ShapeDtypeStruct((N,), jnp.float32),
              jax.ShapeDtypeStruct((N,), jnp.float32)),
    mesh=_MESH,
    compiler_params=pltpu.CompilerParams(needs_layout_passes=False),
    scratch_types=[
        pltpu.VMEM((LANES, 128), jnp.int32),
        pltpu.VMEM((CH,), jnp.float32),
        pltpu.VMEM((CH,), jnp.float32),
        pltpu.VMEM((_NCONST,), jnp.float32),
        pltpu.VMEM((CH,), jnp.float32),
        pltpu.VMEM((CH,), jnp.float32),
        pltpu.VMEM((4 * CH,), jnp.float32),
        pltpu.VMEM((4 * CH,), jnp.float32),
        pltpu.VMEM_SHARED((32, CH), jnp.float32),
        pltpu.VMEM_SHARED((32, CH), jnp.float32),
        pltpu.SemaphoreType.DMA,
    ],
)(_k1_body)


_NB = N // 2 // NCHUNK
_S15 = 15
_H15 = 1 << _S15


def _s15_body(re_hbm, im_hbm, wr_hbm, wi_hbm, out_hbm,
              tre, tim, bre, bim, twr, twi, obuf):
    wid = lax.axis_index("c") * 16 + lax.axis_index("s")
    b0 = wid * _NB
    t0 = pl.multiple_of(((b0 >> _S15) << (_S15 + 1)) + (b0 & (_H15 - 1)), _NB)
    j0 = pl.multiple_of(b0 & (_H15 - 1), _NB)

    pltpu.sync_copy(re_hbm.at[pl.ds(t0, _NB)], tre)
    pltpu.sync_copy(im_hbm.at[pl.ds(t0, _NB)], tim)
    pltpu.sync_copy(re_hbm.at[pl.ds(t0 + _H15, _NB)], bre)
    pltpu.sync_copy(im_hbm.at[pl.ds(t0 + _H15, _NB)], bim)
    pltpu.sync_copy(wr_hbm.at[pl.ds(j0, _NB)], twr)
    pltpu.sync_copy(wi_hbm.at[pl.ds(j0, _NB)], twi)

    def body(o):
        d = pl.ds(o, 16)
        tr = tre[d]
        tii = tim[d]
        br = bre[d]
        bii = bim[d]
        wr = twr[d]
        wi = twi[d]
        xr = wr * br - wi * bii
        xi = wi * br + wr * bii
        tre[d] = tr + xr
        tim[d] = tii + xi
        bre[d] = tr - xr
        bim[d] = tii - xi

    plsc.parallel_loop(0, _NB, 16, unroll=4)(body)

    iota2 = lax.iota(jnp.int32, 16) * 2

    def mk_inter(src_r, src_i):
        def inter(o):
            d = pl.ds(o, 16)
            ix = 2 * o + iota2
            plsc.store_scatter(obuf, [ix], src_r[d])
            plsc.store_scatter(obuf, [ix + 1], src_i[d])
        return inter

    plsc.parallel_loop(0, _NB, 16, unroll=4)(mk_inter(tre, tim))
    pltpu.sync_copy(obuf, out_hbm.at[pl.ds(2 * t0, 2 * _NB)])
    plsc.parallel_loop(0, _NB, 16, unroll=4)(mk_inter(bre, bim))
    pltpu.sync_copy(obuf, out_hbm.at[pl.ds(2 * (t0 + _H15), 2 * _NB)])


_s15 = functools.partial(
    pl.kernel,
    out_type=jax.ShapeDtypeStruct((2 * N,), jnp.float32),
    mesh=_MESH,
    compiler_params=pltpu.CompilerParams(needs_layout_passes=False),
    scratch_types=[pltpu.VMEM((_NB,), jnp.float32)] * 6
    + [pltpu.VMEM((2 * _NB,), jnp.float32)],
)(_s15_body)


def kernel(x):
    re = x[:, 0]
    im = x[:, 1]
    re, im = _k1(re, im, jnp.asarray(_BITREV_IDX), jnp.asarray(_CONSTS),
                 jnp.asarray(_XWR), jnp.asarray(_XWI))
    out = _s15(re, im, jnp.asarray(_WR15), jnp.asarray(_WI15))
    return out.reshape(N, 2)


# speedup vs baseline: 2.7107x; 2.0610x over previous
import functools
import math

import jax
import jax.numpy as jnp
import numpy as np
from jax import lax
from jax.experimental import pallas as pl
from jax.experimental.pallas import tpu as pltpu
from jax.experimental.pallas import tpu_sc as plsc

N = 65536
NCHUNK = 32
CH = 2048
LANES = 16


def _rev_bits(x, nbits):
    r = np.zeros_like(x)
    t = x.copy()
    for _ in range(nbits):
        r = (r << 1) | (t & 1)
        t >>= 1
    return r

_BITREV_IDX = _rev_bits(np.arange(N, dtype=np.int64), 16).reshape(
    NCHUNK, LANES, 128).astype(np.int32)

_lane = np.arange(LANES, dtype=np.int64)
_wr163, _wi163 = [], []
for _s in range(1, 4):
    _h = 1 << _s
    _a = -2.0 * np.pi * (_lane & (_h - 1)) / (2 * _h)
    _wr163.append(np.cos(_a))
    _wi163.append(np.sin(_a))
_LOC_OFF = {}
_twc, _tws = [], []
_o = 0
for _s in range(4, 11):
    _h = 1 << _s
    _a = -2.0 * np.pi * np.arange(_h, dtype=np.float64) / (2 * _h)
    _twc.append(np.cos(_a))
    _tws.append(np.sin(_a))
    _LOC_OFF[_s] = _o
    _o += _h
_NLOC = _o
_WR163_OFF = 0
_WI163_OFF = 48
_TWC_OFF = 96
_TWS_OFF = 96 + _NLOC
_NCONST = 96 + 2 * _NLOC
_CONSTS = np.concatenate(_wr163 + _wi163 + _twc + _tws).astype(np.float32)
assert _CONSTS.shape == (_NCONST,)

_XOFF = {}
_xwr, _xwi = [], []
_o = 0
for _q in range(4):
    _XOFF[_q] = _o
    _n = (1 << _q) * CH
    _a = -2.0 * np.pi * np.arange(_n, dtype=np.float64) / (1 << (12 + _q))
    _xwr.append(np.cos(_a))
    _xwi.append(np.sin(_a))
    _o += _n
_XWR = np.concatenate(_xwr).astype(np.float32)
_XWI = np.concatenate(_xwi).astype(np.float32)

_a15 = -2.0 * np.pi * np.arange(N // 2, dtype=np.float64) / N
_WR15 = np.cos(_a15).astype(np.float32)
_WI15 = np.sin(_a15).astype(np.float32)

_MESH = plsc.VectorSubcoreMesh(
    core_axis_name="c", subcore_axis_name="s", num_cores=2, num_subcores=16)


def _k1_body(re_hbm, im_hbm, idx_hbm, consts_hbm, xwr_hbm, xwi_hbm,
             ore_hbm, oim_hbm,
             idx_v, re_v, im_v, tw_v, pre_v, pim_v, xwr_v, xwi_v,
             shr_re, shr_im, sem):
    sid = lax.axis_index("s")
    wid = lax.axis_index("c") * 16 + sid

    pltpu.sync_copy(idx_hbm.at[wid], idx_v)
    copies = []
    for j in range(LANES):
        d = pl.ds(j * 128, 128)
        copies.append(pltpu.make_async_copy(
            re_hbm.at[idx_v.at[j]], re_v.at[d], sem))
        copies.append(pltpu.make_async_copy(
            im_hbm.at[idx_v.at[j]], im_v.at[d], sem))
    for c in copies:
        c.start()
    pltpu.sync_copy(consts_hbm, tw_v)
    for q in range(4):
        off = pl.multiple_of(_XOFF[q] + (sid & ((1 << q) - 1)) * CH, CH)
        pltpu.sync_copy(xwr_hbm.at[pl.ds(off, CH)],
                        xwr_v.at[pl.ds(q * CH, CH)])
        pltpu.sync_copy(xwi_hbm.at[pl.ds(off, CH)],
                        xwi_v.at[pl.ds(q * CH, CH)])
    for c in copies:
        c.wait()

    iota = lax.iota(jnp.int32, LANES)

    for s in range(0, 4):
        h = 1 << s
        pat = ((iota >> s) << (s + 1)) + (iota & (h - 1))
        if s > 0:
            wr = tw_v[pl.ds(_WR163_OFF + (s - 1) * 16, 16)]
            wi = tw_v[pl.ds(_WI163_OFF + (s - 1) * 16, 16)]

        def body03(k, s=s, h=h, pat=pat,
                   wr=(None if s == 0 else wr), wi=(None if s == 0 else wi)):
            ti = k * 32 + pat
            bi_ = ti + h
            tr = plsc.load_gather(re_v, [ti])
            tii = plsc.load_gather(im_v, [ti])
            br = plsc.load_gather(re_v, [bi_])
            bii = plsc.load_gather(im_v, [bi_])
            if s == 0:
                xr, xi = br, bii
            else:
                xr = wr * br - wi * bii
                xi = wi * br + wr * bii
            plsc.store_scatter(re_v, [ti], tr + xr)
            plsc.store_scatter(im_v, [ti], tii + xi)
            plsc.store_scatter(re_v, [bi_], tr - xr)
            plsc.store_scatter(im_v, [bi_], tii - xi)

        plsc.parallel_loop(0, 64, 1, unroll=4)(body03)

    for s in (4, 6, 8):
        h = 1 << s

        def body4(q, s=s, h=h):
            r = q & (h - 1)
            i0 = ((q >> s) << (s + 2)) + r
            wr = tw_v[pl.ds(_TWC_OFF + _LOC_OFF[s] + r, 16)]
            wi = tw_v[pl.ds(_TWS_OFF + _LOC_OFF[s] + r, 16)]
            u0r = tw_v[pl.ds(_TWC_OFF + _LOC_OFF[s + 1] + r, 16)]
            u0i = tw_v[pl.ds(_TWS_OFF + _LOC_OFF[s + 1] + r, 16)]
            u1r = tw_v[pl.ds(_TWC_OFF + _LOC_OFF[s + 1] + r + h, 16)]
            u1i = tw_v[pl.ds(_TWS_OFF + _LOC_OFF[s + 1] + r + h, 16)]
            d0 = pl.ds(i0, 16)
            d1 = pl.ds(i0 + h, 16)
            d2 = pl.ds(i0 + 2 * h, 16)
            d3 = pl.ds(i0 + 3 * h, 16)
            a0r, a0i = re_v[d0], im_v[d0]
            a1r, a1i = re_v[d1], im_v[d1]
            a2r, a2i = re_v[d2], im_v[d2]
            a3r, a3i = re_v[d3], im_v[d3]
            x1r = wr * a1r - wi * a1i
            x1i = wi * a1r + wr * a1i
            b0r, b0i = a0r + x1r, a0i + x1i
            b1r, b1i = a0r - x1r, a0i - x1i
            x3r = wr * a3r - wi * a3i
            x3i = wi * a3r + wr * a3i
            b2r, b2i = a2r + x3r, a2i + x3i
            b3r, b3i = a2r - x3r, a2i - x3i
            y2r = u0r * b2r - u0i * b2i
            y2i = u0i * b2r + u0r * b2i
            re_v[d0], im_v[d0] = b0r + y2r, b0i + y2i
            re_v[d2], im_v[d2] = b0r - y2r, b0i - y2i
            y3r = u1r * b3r - u1i * b3i
            y3i = u1i * b3r + u1r * b3i
            re_v[d1], im_v[d1] = b1r + y3r, b1i + y3i
            re_v[d3], im_v[d3] = b1r - y3r, b1i - y3i

        plsc.parallel_loop(0, CH // 4, 16, unroll=2)(body4)

    s10, h10 = 10, 1 << 10

    def body10(b):
        r = b & (h10 - 1)
        t0 = ((b >> s10) << (s10 + 1)) + r
        b0 = t0 + h10
        wr = tw_v[pl.ds(_TWC_OFF + _LOC_OFF[s10] + r, 16)]
        wi = tw_v[pl.ds(_TWS_OFF + _LOC_OFF[s10] + r, 16)]
        tr = re_v[pl.ds(t0, 16)]
        tii = im_v[pl.ds(t0, 16)]
        br = re_v[pl.ds(b0, 16)]
        bii = im_v[pl.ds(b0, 16)]
        xr = wr * br - wi * bii
        xi = wi * br + wr * bii
        re_v[pl.ds(t0, 16)] = tr + xr
        im_v[pl.ds(t0, 16)] = tii + xi
        re_v[pl.ds(b0, 16)] = tr - xr
        im_v[pl.ds(b0, 16)] = tii - xi

    plsc.parallel_loop(0, CH // 2, 16, unroll=4)(body10)

    pltpu.sync_copy(re_v, shr_re.at[sid])
    pltpu.sync_copy(im_v, shr_im.at[sid])
    plsc.subcore_barrier()

    for q in range(4):
        psid = sid ^ (1 << q)
        b = q & 1
        pltpu.sync_copy(shr_re.at[b * 16 + psid], pre_v)
        pltpu.sync_copy(shr_im.at[b * 16 + psid], pim_v)
        mt = (((sid >> q) & 1) ^ 1).astype(jnp.float32)
        pt = 1.0 - mt
        sign = 2.0 * mt - 1.0

        def bodyx(o, q=q, mt=mt, pt=pt, sign=sign):
            d = pl.ds(o, 16)
            mr = re_v[d]
            mi = im_v[d]
            pr = pre_v[d]
            pi = pim_v[d]
            wr = xwr_v[pl.ds(q * CH + o, 16)]
            wi = xwi_v[pl.ds(q * CH + o, 16)]
            tr = mt * mr + pt * pr
            tii = mt * mi + pt * pi
            br = mt * pr + pt * mr
            bii = mt * pi + pt * mi
            xr = wr * br - wi * bii
            xi = wi * br + wr * bii
            re_v[d] = tr + sign * xr
            im_v[d] = tii + sign * xi

        plsc.parallel_loop(0, CH, 16, unroll=4)(bodyx)
        if q < 3:
            nb = (q + 1) & 1
            pltpu.sync_copy(re_v, shr_re.at[nb * 16 + sid])
            pltpu.sync_copy(im_v, shr_im.at[nb * 16 + sid])
            plsc.subcore_barrier()

    base = pl.multiple_of(wid * CH, CH)
    pltpu.sync_copy(re_v, ore_hbm.at[pl.ds(base, CH)])
    pltpu.sync_copy(im_v, oim_hbm.at[pl.ds(base, CH)])


_k1 = functools.partial(
    pl.kernel,
    out_type=(jax.ShapeDtypeStruct((N,), jnp.float32),
              jax.ShapeDtypeStruct((N,), jnp.float32)),
    mesh=_MESH,
    compiler_params=pltpu.CompilerParams(needs_layout_passes=False),
    scratch_types=[
        pltpu.VMEM((LANES, 128), jnp.int32),
        pltpu.VMEM((CH,), jnp.float32),
        pltpu.VMEM((CH,), jnp.float32),
        pltpu.VMEM((_NCONST,), jnp.float32),
        pltpu.VMEM((CH,), jnp.float32),
        pltpu.VMEM((CH,), jnp.float32),
        pltpu.VMEM((4 * CH,), jnp.float32),
        pltpu.VMEM((4 * CH,), jnp.float32),
        pltpu.VMEM_SHARED((32, CH), jnp.float32),
        pltpu.VMEM_SHARED((32, CH), jnp.float32),
        pltpu.SemaphoreType.DMA,
    ],
)(_k1_body)


_NB = N // 2 // NCHUNK
_S15 = 15
_H15 = 1 << _S15


def _s15_body(re_hbm, im_hbm, wr_hbm, wi_hbm, ore_hbm, oim_hbm,
              tre, tim, bre, bim, twr, twi):
    wid = lax.axis_index("c") * 16 + lax.axis_index("s")
    b0 = wid * _NB
    t0 = pl.multiple_of(((b0 >> _S15) << (_S15 + 1)) + (b0 & (_H15 - 1)), _NB)
    j0 = pl.multiple_of(b0 & (_H15 - 1), _NB)

    pltpu.sync_copy(re_hbm.at[pl.ds(t0, _NB)], tre)
    pltpu.sync_copy(im_hbm.at[pl.ds(t0, _NB)], tim)
    pltpu.sync_copy(re_hbm.at[pl.ds(t0 + _H15, _NB)], bre)
    pltpu.sync_copy(im_hbm.at[pl.ds(t0 + _H15, _NB)], bim)
    pltpu.sync_copy(wr_hbm.at[pl.ds(j0, _NB)], twr)
    pltpu.sync_copy(wi_hbm.at[pl.ds(j0, _NB)], twi)

    def body(o):
        d = pl.ds(o, 16)
        tr = tre[d]
        tii = tim[d]
        br = bre[d]
        bii = bim[d]
        wr = twr[d]
        wi = twi[d]
        xr = wr * br - wi * bii
        xi = wi * br + wr * bii
        tre[d] = tr + xr
        tim[d] = tii + xi
        bre[d] = tr - xr
        bim[d] = tii - xi

    plsc.parallel_loop(0, _NB, 16, unroll=4)(body)

    pltpu.sync_copy(tre, ore_hbm.at[pl.ds(t0, _NB)])
    pltpu.sync_copy(tim, oim_hbm.at[pl.ds(t0, _NB)])
    pltpu.sync_copy(bre, ore_hbm.at[pl.ds(t0 + _H15, _NB)])
    pltpu.sync_copy(bim, oim_hbm.at[pl.ds(t0 + _H15, _NB)])


_s15 = functools.partial(
    pl.kernel,
    out_type=(jax.ShapeDtypeStruct((N,), jnp.float32),
              jax.ShapeDtypeStruct((N,), jnp.float32)),
    mesh=_MESH,
    scratch_types=[pltpu.VMEM((_NB,), jnp.float32)] * 6,
)(_s15_body)


def kernel(x):
    re = x[:, 0]
    im = x[:, 1]
    re, im = _k1(re, im, jnp.asarray(_BITREV_IDX), jnp.asarray(_CONSTS),
                 jnp.asarray(_XWR), jnp.asarray(_XWI))
    re, im = _s15(re, im, jnp.asarray(_WR15), jnp.asarray(_WI15))
    return jnp.stack((re, im), axis=-1)


# speedup vs baseline: 2.8063x; 1.0353x over previous
import functools
import math

import jax
import jax.numpy as jnp
import numpy as np
from jax import lax
from jax.experimental import pallas as pl
from jax.experimental.pallas import tpu as pltpu
from jax.experimental.pallas import tpu_sc as plsc

N = 65536
NCHUNK = 32
CH = 2048
LANES = 16


def _rev_bits(x, nbits):
    r = np.zeros_like(x)
    t = x.copy()
    for _ in range(nbits):
        r = (r << 1) | (t & 1)
        t >>= 1
    return r

_BITREV_IDX = _rev_bits(np.arange(N, dtype=np.int64), 16).reshape(
    NCHUNK, LANES, 128).astype(np.int32)

_lane = np.arange(LANES, dtype=np.int64)
_wr163, _wi163 = [], []
for _s in range(1, 4):
    _h = 1 << _s
    _a = -2.0 * np.pi * (_lane & (_h - 1)) / (2 * _h)
    _wr163.append(np.cos(_a))
    _wi163.append(np.sin(_a))
_LOC_OFF = {}
_twc, _tws = [], []
_o = 0
for _s in range(4, 11):
    _h = 1 << _s
    _a = -2.0 * np.pi * np.arange(_h, dtype=np.float64) / (2 * _h)
    _twc.append(np.cos(_a))
    _tws.append(np.sin(_a))
    _LOC_OFF[_s] = _o
    _o += _h
_NLOC = _o
_WR163_OFF = 0
_WI163_OFF = 48
_TWC_OFF = 96
_TWS_OFF = 96 + _NLOC
_NCONST = 96 + 2 * _NLOC
_CONSTS = np.concatenate(_wr163 + _wi163 + _twc + _tws).astype(np.float32)
assert _CONSTS.shape == (_NCONST,)

_XOFF = {}
_xwr, _xwi = [], []
_o = 0
for _q in range(4):
    _XOFF[_q] = _o
    _n = (1 << _q) * CH
    _a = -2.0 * np.pi * np.arange(_n, dtype=np.float64) / (1 << (12 + _q))
    _xwr.append(np.cos(_a))
    _xwi.append(np.sin(_a))
    _o += _n
_XWR = np.concatenate(_xwr).astype(np.float32)
_XWI = np.concatenate(_xwi).astype(np.float32)

_a15 = -2.0 * np.pi * np.arange(N // 2, dtype=np.float64) / N
_WR15 = np.cos(_a15).astype(np.float32)
_WI15 = np.sin(_a15).astype(np.float32)

_MESH = plsc.VectorSubcoreMesh(
    core_axis_name="c", subcore_axis_name="s", num_cores=2, num_subcores=16)


def _k1_body(re_hbm, im_hbm, idx_hbm, consts_hbm, xwr_hbm, xwi_hbm,
             ore_hbm, oim_hbm,
             idx_v, re_v, im_v, tw_v, pre_v, pim_v, xwr_v, xwi_v,
             shr_re, shr_im, sem):
    sid = lax.axis_index("s")
    wid = lax.axis_index("c") * 16 + sid

    pltpu.sync_copy(idx_hbm.at[wid], idx_v)
    copies = []
    for j in range(LANES):
        d = pl.ds(j * 128, 128)
        copies.append(pltpu.make_async_copy(
            re_hbm.at[idx_v.at[j]], re_v.at[d], sem))
        copies.append(pltpu.make_async_copy(
            im_hbm.at[idx_v.at[j]], im_v.at[d], sem))
    for c in copies:
        c.start()
    pltpu.sync_copy(consts_hbm, tw_v)
    for q in range(4):
        off = pl.multiple_of(_XOFF[q] + (sid & ((1 << q) - 1)) * CH, CH)
        pltpu.sync_copy(xwr_hbm.at[pl.ds(off, CH)],
                        xwr_v.at[pl.ds(q * CH, CH)])
        pltpu.sync_copy(xwi_hbm.at[pl.ds(off, CH)],
                        xwi_v.at[pl.ds(q * CH, CH)])
    for c in copies:
        c.wait()

    iota = lax.iota(jnp.int32, LANES)

    for s in range(0, 4):
        h = 1 << s
        pat = ((iota >> s) << (s + 1)) + (iota & (h - 1))
        if s > 0:
            wr = tw_v[pl.ds(_WR163_OFF + (s - 1) * 16, 16)]
            wi = tw_v[pl.ds(_WI163_OFF + (s - 1) * 16, 16)]

        def body03(k, s=s, h=h, pat=pat,
                   wr=(None if s == 0 else wr), wi=(None if s == 0 else wi)):
            ti = k * 32 + pat
            bi_ = ti + h
            tr = plsc.load_gather(re_v, [ti])
            tii = plsc.load_gather(im_v, [ti])
            br = plsc.load_gather(re_v, [bi_])
            bii = plsc.load_gather(im_v, [bi_])
            if s == 0:
                xr, xi = br, bii
            else:
                xr = wr * br - wi * bii
                xi = wi * br + wr * bii
            plsc.store_scatter(re_v, [ti], tr + xr)
            plsc.store_scatter(im_v, [ti], tii + xi)
            plsc.store_scatter(re_v, [bi_], tr - xr)
            plsc.store_scatter(im_v, [bi_], tii - xi)

        plsc.parallel_loop(0, 64, 1, unroll=8)(body03)

    for s in (4, 6, 8):
        h = 1 << s

        def body4(q, s=s, h=h):
            r = q & (h - 1)
            i0 = ((q >> s) << (s + 2)) + r
            wr = tw_v[pl.ds(_TWC_OFF + _LOC_OFF[s] + r, 16)]
            wi = tw_v[pl.ds(_TWS_OFF + _LOC_OFF[s] + r, 16)]
            u0r = tw_v[pl.ds(_TWC_OFF + _LOC_OFF[s + 1] + r, 16)]
            u0i = tw_v[pl.ds(_TWS_OFF + _LOC_OFF[s + 1] + r, 16)]
            u1r = tw_v[pl.ds(_TWC_OFF + _LOC_OFF[s + 1] + r + h, 16)]
            u1i = tw_v[pl.ds(_TWS_OFF + _LOC_OFF[s + 1] + r + h, 16)]
            d0 = pl.ds(i0, 16)
            d1 = pl.ds(i0 + h, 16)
            d2 = pl.ds(i0 + 2 * h, 16)
            d3 = pl.ds(i0 + 3 * h, 16)
            a0r, a0i = re_v[d0], im_v[d0]
            a1r, a1i = re_v[d1], im_v[d1]
            a2r, a2i = re_v[d2], im_v[d2]
            a3r, a3i = re_v[d3], im_v[d3]
            x1r = wr * a1r - wi * a1i
            x1i = wi * a1r + wr * a1i
            b0r, b0i = a0r + x1r, a0i + x1i
            b1r, b1i = a0r - x1r, a0i - x1i
            x3r = wr * a3r - wi * a3i
            x3i = wi * a3r + wr * a3i
            b2r, b2i = a2r + x3r, a2i + x3i
            b3r, b3i = a2r - x3r, a2i - x3i
            y2r = u0r * b2r - u0i * b2i
            y2i = u0i * b2r + u0r * b2i
            re_v[d0], im_v[d0] = b0r + y2r, b0i + y2i
            re_v[d2], im_v[d2] = b0r - y2r, b0i - y2i
            y3r = u1r * b3r - u1i * b3i
            y3i = u1i * b3r + u1r * b3i
            re_v[d1], im_v[d1] = b1r + y3r, b1i + y3i
            re_v[d3], im_v[d3] = b1r - y3r, b1i - y3i

        plsc.parallel_loop(0, CH // 4, 16, unroll=2)(body4)

    s10, h10 = 10, 1 << 10

    def body10(b):
        r = b & (h10 - 1)
        t0 = ((b >> s10) << (s10 + 1)) + r
        b0 = t0 + h10
        wr = tw_v[pl.ds(_TWC_OFF + _LOC_OFF[s10] + r, 16)]
        wi = tw_v[pl.ds(_TWS_OFF + _LOC_OFF[s10] + r, 16)]
        tr = re_v[pl.ds(t0, 16)]
        tii = im_v[pl.ds(t0, 16)]
        br = re_v[pl.ds(b0, 16)]
        bii = im_v[pl.ds(b0, 16)]
        xr = wr * br - wi * bii
        xi = wi * br + wr * bii
        re_v[pl.ds(t0, 16)] = tr + xr
        im_v[pl.ds(t0, 16)] = tii + xi
        re_v[pl.ds(b0, 16)] = tr - xr
        im_v[pl.ds(b0, 16)] = tii - xi

    plsc.parallel_loop(0, CH // 2, 16, unroll=8)(body10)

    pltpu.sync_copy(re_v, shr_re.at[sid])
    pltpu.sync_copy(im_v, shr_im.at[sid])
    plsc.subcore_barrier()

    for q in range(4):
        psid = sid ^ (1 << q)
        b = q & 1
        pltpu.sync_copy(shr_re.at[b * 16 + psid], pre_v)
        pltpu.sync_copy(shr_im.at[b * 16 + psid], pim_v)
        mt = (((sid >> q) & 1) ^ 1).astype(jnp.float32)
        pt = 1.0 - mt
        sign = 2.0 * mt - 1.0

        def bodyx(o, q=q, mt=mt, pt=pt, sign=sign):
            d = pl.ds(o, 16)
            mr = re_v[d]
            mi = im_v[d]
            pr = pre_v[d]
            pi = pim_v[d]
            wr = xwr_v[pl.ds(q * CH + o, 16)]
            wi = xwi_v[pl.ds(q * CH + o, 16)]
            tr = mt * mr + pt * pr
            tii = mt * mi + pt * pi
            br = mt * pr + pt * mr
            bii = mt * pi + pt * mi
            xr = wr * br - wi * bii
            xi = wi * br + wr * bii
            re_v[d] = tr + sign * xr
            im_v[d] = tii + sign * xi

        plsc.parallel_loop(0, CH, 16, unroll=8)(bodyx)
        if q < 3:
            nb = (q + 1) & 1
            pltpu.sync_copy(re_v, shr_re.at[nb * 16 + sid])
            pltpu.sync_copy(im_v, shr_im.at[nb * 16 + sid])
            plsc.subcore_barrier()

    base = pl.multiple_of(wid * CH, CH)
    pltpu.sync_copy(re_v, ore_hbm.at[pl.ds(base, CH)])
    pltpu.sync_copy(im_v, oim_hbm.at[pl.ds(base, CH)])


_k1 = functools.partial(
    pl.kernel,
    out_type=(jax.ShapeDtypeStruct((N,), jnp.float32),
              jax.ShapeDtypeStruct((N,), jnp.float32)),
    mesh=_MESH,
    compiler_params=pltpu.CompilerParams(needs_layout_passes=False),
    scratch_types=[
        pltpu.VMEM((LANES, 128), jnp.int32),
        pltpu.VMEM((CH,), jnp.float32),
        pltpu.VMEM((CH,), jnp.float32),
        pltpu.VMEM((_NCONST,), jnp.float32),
        pltpu.VMEM((CH,), jnp.float32),
        pltpu.VMEM((CH,), jnp.float32),
        pltpu.VMEM((4 * CH,), jnp.float32),
        pltpu.VMEM((4 * CH,), jnp.float32),
        pltpu.VMEM_SHARED((32, CH), jnp.float32),
        pltpu.VMEM_SHARED((32, CH), jnp.float32),
        pltpu.SemaphoreType.DMA,
    ],
)(_k1_body)


_NB = N // 2 // NCHUNK
_S15 = 15
_H15 = 1 << _S15


def _s15_body(re_hbm, im_hbm, wr_hbm, wi_hbm, ore_hbm, oim_hbm,
              tre, tim, bre, bim, twr, twi, sem):
    wid = lax.axis_index("c") * 16 + lax.axis_index("s")
    b0 = wid * _NB
    t0 = pl.multiple_of(((b0 >> _S15) << (_S15 + 1)) + (b0 & (_H15 - 1)), _NB)
    j0 = pl.multiple_of(b0 & (_H15 - 1), _NB)

    copies = [
        pltpu.make_async_copy(re_hbm.at[pl.ds(t0, _NB)], tre, sem),
        pltpu.make_async_copy(im_hbm.at[pl.ds(t0, _NB)], tim, sem),
        pltpu.make_async_copy(re_hbm.at[pl.ds(t0 + _H15, _NB)], bre, sem),
        pltpu.make_async_copy(im_hbm.at[pl.ds(t0 + _H15, _NB)], bim, sem),
        pltpu.make_async_copy(wr_hbm.at[pl.ds(j0, _NB)], twr, sem),
        pltpu.make_async_copy(wi_hbm.at[pl.ds(j0, _NB)], twi, sem),
    ]
    for c in copies:
        c.start()
    for c in copies:
        c.wait()

    def body(o):
        d = pl.ds(o, 16)
        tr = tre[d]
        tii = tim[d]
        br = bre[d]
        bii = bim[d]
        wr = twr[d]
        wi = twi[d]
        xr = wr * br - wi * bii
        xi = wi * br + wr * bii
        tre[d] = tr + xr
        tim[d] = tii + xi
        bre[d] = tr - xr
        bim[d] = tii - xi

    plsc.parallel_loop(0, _NB, 16, unroll=8)(body)

    pltpu.sync_copy(tre, ore_hbm.at[pl.ds(t0, _NB)])
    pltpu.sync_copy(tim, oim_hbm.at[pl.ds(t0, _NB)])
    pltpu.sync_copy(bre, ore_hbm.at[pl.ds(t0 + _H15, _NB)])
    pltpu.sync_copy(bim, oim_hbm.at[pl.ds(t0 + _H15, _NB)])


_s15 = functools.partial(
    pl.kernel,
    out_type=(jax.ShapeDtypeStruct((N,), jnp.float32),
              jax.ShapeDtypeStruct((N,), jnp.float32)),
    mesh=_MESH,
    scratch_types=[pltpu.VMEM((_NB,), jnp.float32)] * 6
    + [pltpu.SemaphoreType.DMA],
)(_s15_body)


def kernel(x):
    re = x[:, 0]
    im = x[:, 1]
    re, im = _k1(re, im, jnp.asarray(_BITREV_IDX), jnp.asarray(_CONSTS),
                 jnp.asarray(_XWR), jnp.asarray(_XWI))
    re, im = _s15(re, im, jnp.asarray(_WR15), jnp.asarray(_WI15))
    return jnp.stack((re, im), axis=-1)
